# baseline (speedup 1.0000x reference)
import functools

import jax
import jax.numpy as jnp
from jax.experimental import pallas as pl
from jax.experimental.pallas import tpu as pltpu

_NUM_EXPERTS = 16
_TOP_K = 2
_BLK = 1024
_H = 2048


def _chunk_copy(x_hbm, xbuf, sems, step, slot):
    return pltpu.make_async_copy(
        x_hbm.at[pl.ds(step * _BLK, _BLK)], xbuf.at[slot], sems.at[slot]
    )


def _gate_kernel(x_hbm, w_ref, s_ref, i_ref, xbuf, sems):
    n = x_hbm.shape[0] // _BLK
    w = w_ref[...]
    _chunk_copy(x_hbm, xbuf, sems, 0, 0).start()

    def body(step, _):
        slot = jax.lax.rem(step, 2)
        nxt = jax.lax.rem(step + 1, 2)

        @pl.when(step + 1 < n)
        def _():
            _chunk_copy(x_hbm, xbuf, sems, step + 1, nxt).start()

        _chunk_copy(x_hbm, xbuf, sems, step, slot).wait()

        lt = jax.lax.dot_general(
            w, xbuf[slot], (((1,), (1,)), ((), ())),
            preferred_element_type=jnp.float32,
        )
        m = jnp.max(lt, axis=0, keepdims=True)
        row = jax.lax.broadcasted_iota(jnp.int32, lt.shape, 0)
        i1 = jnp.min(jnp.where(lt == m, row, _NUM_EXPERTS), axis=0, keepdims=True)
        masked = jnp.where(row == i1, -jnp.inf, lt)
        l2 = jnp.max(masked, axis=0, keepdims=True)
        i2 = jnp.min(jnp.where(masked == l2, row, _NUM_EXPERTS), axis=0, keepdims=True)
        z = jnp.sum(jnp.exp(lt - m), axis=0, keepdims=True)

        v1 = 1.0 / z
        t = jnp.exp(jnp.exp(l2 - m) / z - v1)
        d = 1.0 + t
        base = step * _BLK
        s_ref[pl.ds(base, _BLK), :] = jnp.concatenate([1.0 / d, t / d], axis=0).T
        i_ref[pl.ds(base, _BLK), :] = jnp.concatenate([i1, i2], axis=0).T
        return 0

    jax.lax.fori_loop(0, n, body, 0)


@functools.partial(jax.jit, static_argnums=())
def kernel(x, W_gate):
    b, s, h = x.shape
    rows = b * s
    x_flat = x.reshape(rows, h)
    scores, idx = pl.pallas_call(
        _gate_kernel,
        in_specs=[
            pl.BlockSpec(memory_space=pl.ANY),
            pl.BlockSpec(memory_space=pltpu.VMEM),
        ],
        out_specs=[
            pl.BlockSpec(memory_space=pltpu.VMEM),
            pl.BlockSpec(memory_space=pltpu.VMEM),
        ],
        out_shape=[
            jax.ShapeDtypeStruct((rows, _TOP_K), jnp.float32),
            jax.ShapeDtypeStruct((rows, _TOP_K), jnp.int32),
        ],
        scratch_shapes=[
            pltpu.VMEM((2, _BLK, _H), jnp.float32),
            pltpu.SemaphoreType.DMA((2,)),
        ],
        compiler_params=pltpu.CompilerParams(
            dimension_semantics=(),
        ),
    )(x_flat, W_gate)
    aux_loss = jnp.array(0.0, dtype=jnp.float32)
    return (scores, idx, aux_loss)


# speedup vs baseline: 1.2255x; 1.2255x over previous
import functools

import jax
import jax.numpy as jnp
from jax.experimental import pallas as pl
from jax.experimental.pallas import tpu as pltpu

_NUM_EXPERTS = 16
_TOP_K = 2
_BLK = 1024


def _gate_kernel(x_ref, w_ref, o_ref):
    lt = jax.lax.dot_general(
        w_ref[...], x_ref[...], (((1,), (1,)), ((), ())),
        preferred_element_type=jnp.float32,
    )
    m = jnp.max(lt, axis=0, keepdims=True)
    row = jax.lax.broadcasted_iota(jnp.int32, lt.shape, 0)
    i1 = jnp.min(jnp.where(lt == m, row, _NUM_EXPERTS), axis=0, keepdims=True)
    masked = jnp.where(row == i1, -jnp.inf, lt)
    l2 = jnp.max(masked, axis=0, keepdims=True)
    i2 = jnp.min(jnp.where(masked == l2, row, _NUM_EXPERTS), axis=0, keepdims=True)
    z = jnp.sum(jnp.exp(lt - m), axis=0, keepdims=True)

    v1 = 1.0 / z
    t = jnp.exp(jnp.exp(l2 - m) / z - v1)
    d = 1.0 + t
    p1 = 1.0 / d
    p2 = t / d
    b1 = jax.lax.bitcast_convert_type(i1, jnp.float32)
    b2 = jax.lax.bitcast_convert_type(i2, jnp.float32)
    o_ref[...] = jnp.concatenate([p1, p2, b1, b2, p1, p1, p1, p1], axis=0)


@functools.partial(jax.jit, static_argnums=())
def kernel(x, W_gate):
    b, s, h = x.shape
    rows = b * s
    x_flat = x.reshape(rows, h)
    grid = (rows // _BLK,)
    packed = pl.pallas_call(
        _gate_kernel,
        grid=grid,
        in_specs=[
            pl.BlockSpec((_BLK, h), lambda i: (i, 0)),
            pl.BlockSpec((_NUM_EXPERTS, h), lambda i: (0, 0)),
        ],
        out_specs=pl.BlockSpec((8, _BLK), lambda i: (0, i)),
        out_shape=jax.ShapeDtypeStruct((8, rows), jnp.float32),
        compiler_params=pltpu.CompilerParams(
            dimension_semantics=("arbitrary",),
        ),
    )(x_flat, W_gate)
    scores = packed[0:2].T
    idx = jax.lax.bitcast_convert_type(packed[2:4], jnp.int32).T
    aux_loss = jnp.array(0.0, dtype=jnp.float32)
    return (scores, idx, aux_loss)
